# baseline (speedup 1.0000x reference)
import functools

import jax
import jax.numpy as jnp
from jax import lax
from jax.experimental import pallas as pl
from jax.experimental.pallas import tpu as pltpu
from jax.experimental.pallas import tpu_sc as plsc

D = 128
ED = 16
H = 128
VL = 16


def _pack_halves(x):
    lo = x[:, :H // 2].astype(jnp.bfloat16).astype(jnp.float32)
    hi = x[:, H // 2:].astype(jnp.bfloat16).astype(jnp.float32)
    bl = jax.lax.bitcast_convert_type(lo, jnp.int32)
    bh = jax.lax.bitcast_convert_type(hi, jnp.int32)
    return ((bl >> 16) & 0xFFFF) | (bh & jnp.int32(-65536))


def _node_stage_body(h_ref, wn1_ref, bn1_ref, wn2_ref, bn2_ref, wi_ref,
                     wj_ref, bm1_ref, a_ref, b_ref):
    hb = h_ref[...]
    u = jnp.maximum(
        jnp.dot(hb, wn1_ref[...], preferred_element_type=jnp.float32)
        + bn1_ref[...], 0.0)
    u = jnp.dot(u, wn2_ref[...], preferred_element_type=jnp.float32) + bn2_ref[...]
    a_ref[...] = (jnp.dot(u, wi_ref[...], preferred_element_type=jnp.float32)
                  + bm1_ref[...])
    b_ref[...] = jnp.dot(u, wj_ref[...], preferred_element_type=jnp.float32)


def _node_stage(h, wn1, bn1, wn2, bn2, wi, wj, bm1, block):
    n = h.shape[0]
    grid = (n // block,)
    full = lambda s: pl.BlockSpec(s, lambda i: (0,) * len(s))
    return pl.pallas_call(
        _node_stage_body,
        grid=grid,
        in_specs=[
            pl.BlockSpec((block, D), lambda i: (i, 0)),
            full((D, 2 * D)), full((1, 2 * D)), full((2 * D, D)), full((1, D)),
            full((D, H)), full((D, H)), full((1, H)),
        ],
        out_specs=[pl.BlockSpec((block, H), lambda i: (i, 0)),
                   pl.BlockSpec((block, H), lambda i: (i, 0))],
        out_shape=[jax.ShapeDtypeStruct((n, H), jnp.float32),
                   jax.ShapeDtypeStruct((n, H), jnp.float32)],
    )(h, wn1, bn1.reshape(1, -1), wn2, bn2.reshape(1, -1), wi, wj,
      bm1.reshape(1, -1))


def _edge_stage_body(ea_ref, w1_ref, b1_ref, w2_ref, b2_ref, c_ref):
    def half(x):
        t = jnp.maximum(
            jnp.dot(x, w1_ref[...], preferred_element_type=jnp.float32)
            + b1_ref[...], 0.0)
        return _pack_halves(
            jnp.dot(t, w2_ref[...], preferred_element_type=jnp.float32)
            + b2_ref[...])
    ea = ea_ref[...]
    c_ref[...] = jnp.concatenate(
        [half(ea[:, :ED]), half(ea[:, ED:])], axis=1)


def _edge_stage(ea2, w1, b1, w2m, b2m, block):
    e2 = ea2.shape[0]
    full = lambda s: pl.BlockSpec(s, lambda i: (0,) * len(s))
    return pl.pallas_call(
        _edge_stage_body,
        grid=(e2 // block,),
        in_specs=[
            pl.BlockSpec((block, 2 * ED), lambda i: (i, 0)),
            full((ED, 2 * ED)), full((1, 2 * ED)), full((2 * ED, H)),
            full((1, H)),
        ],
        out_specs=pl.BlockSpec((block, H), lambda i: (i, 0)),
        out_shape=jax.ShapeDtypeStruct((e2, H), jnp.int32),
    )(ea2, w1, b1.reshape(1, -1), w2m, b2m.reshape(1, -1))


def _post_stage_body(p_ref, cnt_ref, wm2_ref, bm2_ref, g_ref, b_ref,
                     wf_ref, bf_ref, o_ref, *, final):
    s = p_ref[0] + p_ref[1]
    cnt = cnt_ref[0, :, :1] + cnt_ref[1, :, :1]
    sf = (jnp.dot(s, wm2_ref[...], preferred_element_type=jnp.float32)
          + cnt * bm2_ref[...])
    h = sf / jnp.maximum(cnt, 1.0)
    mu = jnp.mean(h, axis=-1, keepdims=True)
    d = h - mu
    var = jnp.mean(d * d, axis=-1, keepdims=True)
    h = d / jnp.sqrt(var + 1e-5) * g_ref[...] + b_ref[...]
    h = jnp.maximum(h, 0.0)
    if final:
        o_ref[...] = (jnp.dot(h, wf_ref[...], preferred_element_type=jnp.float32)
                      + bf_ref[...])
    else:
        o_ref[...] = h


def _post_stage(p, cntp, wm2, bm2, g, b, wf, bf, n, block, final):
    out_w = wf.shape[1] if final else H
    full = lambda s: pl.BlockSpec(s, lambda i: (0,) * len(s))
    return pl.pallas_call(
        functools.partial(_post_stage_body, final=final),
        grid=(n // block,),
        in_specs=[
            pl.BlockSpec((2, block, H), lambda i: (0, i, 0)),
            pl.BlockSpec((2, block, VL), lambda i: (0, i, 0)),
            full((H, H)), full((1, H)), full((1, H)), full((1, H)),
            full(wf.shape), full((1, wf.shape[1])),
        ],
        out_specs=pl.BlockSpec((block, out_w), lambda i: (i, 0)),
        out_shape=jax.ShapeDtypeStruct((n, out_w), jnp.float32),
    )(p, cntp, wm2, bm2.reshape(1, -1), g.reshape(1, -1), b.reshape(1, -1),
      wf, bf.reshape(1, -1))


def _post_node_body(p_ref, cnt_ref, wm2_ref, bm2_ref, g_ref, b_ref,
                    wn1_ref, bn1_ref, wn2_ref, bn2_ref, wi_ref, wj_ref,
                    bm1_ref, a_ref, b2_ref):
    s = p_ref[0] + p_ref[1]
    cnt = cnt_ref[0, :, :1] + cnt_ref[1, :, :1]
    sf = (jnp.dot(s, wm2_ref[...], preferred_element_type=jnp.float32)
          + cnt * bm2_ref[...])
    h = sf / jnp.maximum(cnt, 1.0)
    mu = jnp.mean(h, axis=-1, keepdims=True)
    d = h - mu
    var = jnp.mean(d * d, axis=-1, keepdims=True)
    h = d / jnp.sqrt(var + 1e-5) * g_ref[...] + b_ref[...]
    h = jnp.maximum(h, 0.0)
    u = jnp.maximum(
        jnp.dot(h, wn1_ref[...], preferred_element_type=jnp.float32)
        + bn1_ref[...], 0.0)
    u = (jnp.dot(u, wn2_ref[...], preferred_element_type=jnp.float32)
         + bn2_ref[...])
    a_ref[...] = (jnp.dot(u, wi_ref[...], preferred_element_type=jnp.float32)
                  + bm1_ref[...])
    b2_ref[...] = jnp.dot(u, wj_ref[...], preferred_element_type=jnp.float32)


def _post_node_stage(pp, cntp, wm2, bm2, g, b, wn1, bn1, wn2, bn2, wi, wj,
                     bm1, n, block):
    full = lambda s: pl.BlockSpec(s, lambda i: (0,) * len(s))
    return pl.pallas_call(
        _post_node_body,
        grid=(n // block,),
        in_specs=[
            pl.BlockSpec((2, block, H), lambda i: (0, i, 0)),
            pl.BlockSpec((2, block, VL), lambda i: (0, i, 0)),
            full((H, H)), full((1, H)), full((1, H)), full((1, H)),
            full((D, 2 * D)), full((1, 2 * D)), full((2 * D, D)),
            full((1, D)), full((D, H)), full((D, H)), full((1, H)),
        ],
        out_specs=[pl.BlockSpec((block, H), lambda i: (i, 0)),
                   pl.BlockSpec((block, H), lambda i: (i, 0))],
        out_shape=[jax.ShapeDtypeStruct((n, H), jnp.float32),
                   jax.ShapeDtypeStruct((n, H), jnp.float32)],
    )(pp, cntp, wm2, bm2.reshape(1, -1), g.reshape(1, -1), b.reshape(1, -1),
      wn1, bn1.reshape(1, -1), wn2, bn2.reshape(1, -1), wi, wj,
      bm1.reshape(1, -1))


def _sc_geometry(n, e):
    mesh = plsc.VectorSubcoreMesh(core_axis_name="c", subcore_axis_name="s")
    nc, ns = mesh.num_cores, mesh.num_subcores
    epw = e // (nc * ns)
    k = 80
    rows_pt = ((-(-n // ns)) + k - 1) // k * k
    n_pad = rows_pt * ns
    return mesh, nc, ns, epw, k, epw // k, rows_pt, n_pad, rows_pt // k


def _make_sc_edge_pass(n, e):
    mesh, nc, ns, _, _, _, _, _, _ = _sc_geometry(n, e)
    nw = nc * ns
    ke = 32
    total_ch = e // ke
    max_t = -(-total_ch // nw)
    nquads = (max_t + 4) // 4
    rows_pt = ((-(-n // ns)) + ke - 1) // ke * ke
    n_pad = rows_pt * ns
    nzc = rows_pt // ke

    out_type = [jax.ShapeDtypeStruct((nc, n_pad, H), jnp.float32)]
    scratch = [
        pltpu.VMEM_SHARED((n_pad, H), jnp.float32),
        pltpu.VMEM((4, ke), jnp.int32),
        pltpu.VMEM((4, ke), jnp.int32),
        pltpu.VMEM((ke, H), jnp.float32),
        pltpu.VMEM((ke, H), jnp.float32),
        pltpu.VMEM((ke, H), jnp.float32),
        pltpu.VMEM((ke, H), jnp.float32),
        pltpu.VMEM((ke // 2, H), jnp.int32),
        pltpu.VMEM((ke // 2, H), jnp.int32),
        pltpu.VMEM((ke, H), jnp.float32),
        pltpu.VMEM((ke, H), jnp.float32),
        pltpu.SemaphoreType.DMA,
        pltpu.SemaphoreType.DMA,
        pltpu.SemaphoreType.DMA,
        pltpu.SemaphoreType.DMA,
        pltpu.SemaphoreType.DMA,
        pltpu.SemaphoreType.DMA,
    ]

    def body(a_hbm, b_hbm, c_hbm, dst_hbm, src_hbm, p_hbm,
             s_sh, dir_, sir_, ab0, ab1, bb0, bb1, cb0, cb1, mb0, mb1,
             sem_i0, sem_i1, sem_g0, sem_g1, sem_s0, sem_s1):
        ab = (ab0, ab1)
        bb = (bb0, bb1)
        cb = (cb0, cb1)
        mb = (mb0, mb1)
        sem_i = (sem_i0, sem_i1)
        sem_g = (sem_g0, sem_g1)
        sem_s = (sem_s0, sem_s1)
        c_id = lax.axis_index("c")
        s_id = lax.axis_index("s")
        wid = c_id * ns + s_id
        row0 = s_id * rows_pt
        nt = lax.div(total_ch - wid + (nw - 1), nw)

        zeros16 = jnp.zeros((VL,), jnp.float32)

        def zfill(i, _):
            for j in range(H // VL):
                mb0[i, pl.ds(j * VL, VL)] = zeros16
            return 0
        lax.fori_loop(0, ke, zfill, 0)
        for j in range(nzc):
            pltpu.sync_copy(mb0, s_sh.at[pl.ds(row0 + j * ke, ke)])
        plsc.subcore_barrier()

        def eoff_of(t):
            return pl.multiple_of((wid + nw * t) * ke, 8)

        def issue_idx(t, q, sync=False):
            eoff = eoff_of(t)
            if sync:
                pltpu.sync_copy(dst_hbm.at[pl.ds(eoff, ke)], dir_.at[q])
                pltpu.sync_copy(src_hbm.at[pl.ds(eoff, ke)], sir_.at[q])
            else:
                pltpu.async_copy(dst_hbm.at[pl.ds(eoff, ke)], dir_.at[q],
                                 sem_i[q % 2])
                pltpu.async_copy(src_hbm.at[pl.ds(eoff, ke)], sir_.at[q],
                                 sem_i[q % 2])

        def wait_idx(q):
            pltpu.make_async_copy(dst_hbm.at[pl.ds(0, ke)], dir_.at[q],
                                  sem_i[q % 2]).wait()
            pltpu.make_async_copy(src_hbm.at[pl.ds(0, ke)], sir_.at[q],
                                  sem_i[q % 2]).wait()

        def issue_gathers(t, b, q):
            eoff2 = pl.multiple_of((wid + nw * t) * (ke // 2), 8)
            pltpu.async_copy(a_hbm.at[dir_.at[q]], ab[b], sem_g[b])
            pltpu.async_copy(b_hbm.at[sir_.at[q]], bb[b], sem_g[b])
            pltpu.async_copy(c_hbm.at[pl.ds(eoff2, ke // 2)], cb[b],
                             sem_g[b])

        def wait_gathers(b):
            pltpu.make_async_copy(a_hbm.at[dir_.at[0]], ab[b],
                                  sem_g[b]).wait()
            pltpu.make_async_copy(b_hbm.at[sir_.at[0]], bb[b],
                                  sem_g[b]).wait()
            pltpu.make_async_copy(c_hbm.at[pl.ds(0, ke // 2)], cb[b],
                                  sem_g[b]).wait()

        def wait_scatter(b):
            pltpu.make_async_copy(mb[b], s_sh.at[dir_.at[0]],
                                  sem_s[b]).wait()

        issue_idx(0, 0, sync=True)
        issue_idx(1, 1)
        issue_gathers(0, 0, 0)

        himask = jnp.full((VL,), -65536, jnp.int32)

        def quad(i4, _):
            for qq in range(4):
                t = 4 * i4 + qq
                b = qq % 2

                @pl.when(t < nt)
                def _():
                    @pl.when(t + 1 < nt)
                    def _():
                        wait_idx((qq + 1) % 4)
                        issue_gathers(t + 1, 1 - b, (qq + 1) % 4)
                    wait_gathers(b)

                    @pl.when(t >= 2)
                    def _():
                        wait_scatter(b)

                    def edge2(r2, _):
                        for hh in range(2):
                            r = 2 * r2 + hh
                            for j in range(H // 2 // VL):
                                ci = cb[b][r2, pl.ds(hh * (H // 2) + j * VL,
                                                     VL)]
                                lo = pl.ds(j * VL, VL)
                                hi = pl.ds(H // 2 + j * VL, VL)
                                mb[b][r, lo] = jnp.maximum(
                                    ab[b][r, lo] + bb[b][r, lo]
                                    + plsc.bitcast(ci << 16, jnp.float32),
                                    zeros16)
                                mb[b][r, hi] = jnp.maximum(
                                    ab[b][r, hi] + bb[b][r, hi]
                                    + plsc.bitcast(ci & himask, jnp.float32),
                                    zeros16)
                        return 0
                    lax.fori_loop(0, ke // 2, edge2, 0)

                    pltpu.async_copy(mb[b], s_sh.at[dir_.at[qq]], sem_s[b],
                                     add=True)

                    @pl.when(t + 2 < nt)
                    def _():
                        issue_idx(t + 2, (qq + 2) % 4)
            return 0
        lax.fori_loop(0, nquads, quad, 0)

        wait_scatter(0)
        wait_scatter(1)
        plsc.subcore_barrier()

        for j in range(nzc):
            rows = pl.ds(row0 + j * ke, ke)
            pltpu.sync_copy(s_sh.at[rows], mb0)
            pltpu.sync_copy(mb0, p_hbm.at[c_id, rows])

    return pl.kernel(body, out_type=out_type, mesh=mesh,
                     scratch_types=scratch,
                     compiler_params=pltpu.CompilerParams(
                         needs_layout_passes=False))


def _make_sc_cnt_pass(n, e):
    mesh, nc, ns, epw, k, nch, rows_pt, n_pad, nzc = _sc_geometry(n, e)

    out_type = [jax.ShapeDtypeStruct((nc, n_pad, VL), jnp.float32)]
    scratch = [
        pltpu.VMEM_SHARED((n_pad, VL), jnp.float32),
        pltpu.VMEM((k,), jnp.int32),
        pltpu.VMEM((k,), jnp.int32),
        pltpu.VMEM((k, VL), jnp.float32),
        pltpu.SemaphoreType.DMA,
        pltpu.SemaphoreType.DMA,
    ]

    def body(dst_hbm, cnt_hbm, cnt_sh, di0, di1, ones, sem0, sem1):
        di = (di0, di1)
        sem = (sem0, sem1)
        c_id = lax.axis_index("c")
        s_id = lax.axis_index("s")
        wid = c_id * ns + s_id
        row0 = s_id * rows_pt

        def fill(val):
            def f(i, _):
                ones[i, pl.ds(0, VL)] = jnp.full((VL,), val, jnp.float32)
                return 0
            lax.fori_loop(0, k, f, 0)

        def issue_idx(t, b):
            eoff = pl.multiple_of(wid * epw + t * k, 8)
            pltpu.async_copy(dst_hbm.at[pl.ds(eoff, k)], di[b], sem[b])

        fill(0.0)
        for j in range(nzc):
            pltpu.sync_copy(ones, cnt_sh.at[pl.ds(row0 + j * k, k)])
        fill(1.0)
        plsc.subcore_barrier()

        issue_idx(0, 0)
        issue_idx(1, 1)

        def pair(i2, _):
            for b in (0, 1):
                t = 2 * i2 + b

                @pl.when(t < nch)
                def _():
                    pltpu.make_async_copy(dst_hbm.at[pl.ds(0, k)], di[b],
                                          sem[b]).wait()
                    pltpu.sync_copy(ones, cnt_sh.at[di[b]], add=True)

                    @pl.when(t + 2 < nch)
                    def _():
                        issue_idx(t + 2, b)
            return 0
        lax.fori_loop(0, (nch + 2) // 2, pair, 0)
        plsc.subcore_barrier()

        for j in range(nzc):
            rows = pl.ds(row0 + j * k, k)
            pltpu.sync_copy(cnt_sh.at[rows], ones)
            pltpu.sync_copy(ones, cnt_hbm.at[c_id, rows])

    return pl.kernel(body, out_type=out_type, mesh=mesh,
                     scratch_types=scratch)


def kernel(x, edge_index, edge_attr, Wn1, bn1, Wn2, bn2, We1, be1, We2, be2,
           Wm1, bm1, Wm2, bm2, g, b, Wf, bf):
    n = x.shape[0]
    e = edge_index.shape[1]
    num_layers = Wn1.shape[0]
    src = edge_index[0]
    dst = edge_index[1]

    sc_pass = _make_sc_edge_pass(n, e)
    (cntp,) = _make_sc_cnt_pass(n, e)(dst)

    ea2 = edge_attr.reshape(e // 2, 2 * ED)
    wi = [Wm1[l][:D] for l in range(num_layers)]
    wj = [Wm1[l][D:2 * D] for l in range(num_layers)]
    we = [Wm1[l][2 * D:] for l in range(num_layers)]
    c0 = _edge_stage(ea2, We1[0], be1[0], We2[0] @ we[0], be2[0] @ we[0],
                     block=16000)
    a, bmat = _node_stage(x, Wn1[0], bn1[0], Wn2[0], bn2[0], wi[0], wj[0],
                          bm1[0], block=5000)
    (p,) = sc_pass(a, bmat, c0, dst, src)
    c1 = _edge_stage(ea2, We1[1], be1[1], We2[1] @ we[1], be2[1] @ we[1],
                     block=16000)
    a, bmat = _post_node_stage(p, cntp, Wm2[0], bm2[0], g[0], b[0],
                               Wn1[1], bn1[1], Wn2[1], bn2[1], wi[1], wj[1],
                               bm1[1], n=n, block=5000)
    (p,) = sc_pass(a, bmat, c1, dst, src)
    return _post_stage(p, cntp, Wm2[1], bm2[1], g[1], b[1], Wf, bf,
                       n=n, block=5000, final=True)


# speedup vs baseline: 1.0043x; 1.0043x over previous
import functools

import jax
import jax.numpy as jnp
from jax import lax
from jax.experimental import pallas as pl
from jax.experimental.pallas import tpu as pltpu
from jax.experimental.pallas import tpu_sc as plsc

D = 128
ED = 16
H = 128
VL = 16


def _pack_halves(x):
    lo = x[:, :H // 2].astype(jnp.bfloat16).astype(jnp.float32)
    hi = x[:, H // 2:].astype(jnp.bfloat16).astype(jnp.float32)
    bl = jax.lax.bitcast_convert_type(lo, jnp.int32)
    bh = jax.lax.bitcast_convert_type(hi, jnp.int32)
    return ((bl >> 16) & 0xFFFF) | (bh & jnp.int32(-65536))


def _node_stage_body(h_ref, wn1_ref, bn1_ref, wn2_ref, bn2_ref, wi_ref,
                     wj_ref, bm1_ref, a_ref, b_ref):
    hb = h_ref[...]
    u = jnp.maximum(
        jnp.dot(hb, wn1_ref[...], preferred_element_type=jnp.float32)
        + bn1_ref[...], 0.0)
    u = jnp.dot(u, wn2_ref[...], preferred_element_type=jnp.float32) + bn2_ref[...]
    a_ref[...] = (jnp.dot(u, wi_ref[...], preferred_element_type=jnp.float32)
                  + bm1_ref[...])
    b_ref[...] = jnp.dot(u, wj_ref[...], preferred_element_type=jnp.float32)


def _node_stage(h, wn1, bn1, wn2, bn2, wi, wj, bm1, block):
    n = h.shape[0]
    grid = (n // block,)
    full = lambda s: pl.BlockSpec(s, lambda i: (0,) * len(s))
    return pl.pallas_call(
        _node_stage_body,
        grid=grid,
        in_specs=[
            pl.BlockSpec((block, D), lambda i: (i, 0)),
            full((D, 2 * D)), full((1, 2 * D)), full((2 * D, D)), full((1, D)),
            full((D, H)), full((D, H)), full((1, H)),
        ],
        out_specs=[pl.BlockSpec((block, H), lambda i: (i, 0)),
                   pl.BlockSpec((block, H), lambda i: (i, 0))],
        out_shape=[jax.ShapeDtypeStruct((n, H), jnp.float32),
                   jax.ShapeDtypeStruct((n, H), jnp.float32)],
    )(h, wn1, bn1.reshape(1, -1), wn2, bn2.reshape(1, -1), wi, wj,
      bm1.reshape(1, -1))


def _edge_stage_body(ea_ref, w1_ref, b1_ref, w2_ref, b2_ref, c_ref):
    def half(x):
        t = jnp.maximum(
            jnp.dot(x, w1_ref[...], preferred_element_type=jnp.float32)
            + b1_ref[...], 0.0)
        return _pack_halves(
            jnp.dot(t, w2_ref[...], preferred_element_type=jnp.float32)
            + b2_ref[...])
    ea = ea_ref[...]
    c_ref[...] = jnp.concatenate(
        [half(ea[:, :ED]), half(ea[:, ED:])], axis=1)


def _edge_stage(ea2, w1, b1, w2m, b2m, block):
    e2 = ea2.shape[0]
    full = lambda s: pl.BlockSpec(s, lambda i: (0,) * len(s))
    return pl.pallas_call(
        _edge_stage_body,
        grid=(e2 // block,),
        in_specs=[
            pl.BlockSpec((block, 2 * ED), lambda i: (i, 0)),
            full((ED, 2 * ED)), full((1, 2 * ED)), full((2 * ED, H)),
            full((1, H)),
        ],
        out_specs=pl.BlockSpec((block, H), lambda i: (i, 0)),
        out_shape=jax.ShapeDtypeStruct((e2, H), jnp.int32),
    )(ea2, w1, b1.reshape(1, -1), w2m, b2m.reshape(1, -1))


def _post_stage_body(p_ref, cnt_ref, wm2_ref, bm2_ref, g_ref, b_ref,
                     wf_ref, bf_ref, o_ref, *, final):
    s = p_ref[0] + p_ref[1]
    cnt = cnt_ref[0, :, :1] + cnt_ref[1, :, :1]
    sf = (jnp.dot(s, wm2_ref[...], preferred_element_type=jnp.float32)
          + cnt * bm2_ref[...])
    h = sf / jnp.maximum(cnt, 1.0)
    mu = jnp.mean(h, axis=-1, keepdims=True)
    d = h - mu
    var = jnp.mean(d * d, axis=-1, keepdims=True)
    h = d / jnp.sqrt(var + 1e-5) * g_ref[...] + b_ref[...]
    h = jnp.maximum(h, 0.0)
    if final:
        o_ref[...] = (jnp.dot(h, wf_ref[...], preferred_element_type=jnp.float32)
                      + bf_ref[...])
    else:
        o_ref[...] = h


def _post_stage(p, cntp, wm2, bm2, g, b, wf, bf, n, block, final):
    out_w = wf.shape[1] if final else H
    full = lambda s: pl.BlockSpec(s, lambda i: (0,) * len(s))
    return pl.pallas_call(
        functools.partial(_post_stage_body, final=final),
        grid=(n // block,),
        in_specs=[
            pl.BlockSpec((2, block, H), lambda i: (0, i, 0)),
            pl.BlockSpec((2, block, VL), lambda i: (0, i, 0)),
            full((H, H)), full((1, H)), full((1, H)), full((1, H)),
            full(wf.shape), full((1, wf.shape[1])),
        ],
        out_specs=pl.BlockSpec((block, out_w), lambda i: (i, 0)),
        out_shape=jax.ShapeDtypeStruct((n, out_w), jnp.float32),
    )(p, cntp, wm2, bm2.reshape(1, -1), g.reshape(1, -1), b.reshape(1, -1),
      wf, bf.reshape(1, -1))


def _post_node_body(p_ref, cnt_ref, wm2_ref, bm2_ref, g_ref, b_ref,
                    wn1_ref, bn1_ref, wn2_ref, bn2_ref, wi_ref, wj_ref,
                    bm1_ref, a_ref, b2_ref):
    s = p_ref[0] + p_ref[1]
    cnt = cnt_ref[0, :, :1] + cnt_ref[1, :, :1]
    sf = (jnp.dot(s, wm2_ref[...], preferred_element_type=jnp.float32)
          + cnt * bm2_ref[...])
    h = sf / jnp.maximum(cnt, 1.0)
    mu = jnp.mean(h, axis=-1, keepdims=True)
    d = h - mu
    var = jnp.mean(d * d, axis=-1, keepdims=True)
    h = d / jnp.sqrt(var + 1e-5) * g_ref[...] + b_ref[...]
    h = jnp.maximum(h, 0.0)
    u = jnp.maximum(
        jnp.dot(h, wn1_ref[...], preferred_element_type=jnp.float32)
        + bn1_ref[...], 0.0)
    u = (jnp.dot(u, wn2_ref[...], preferred_element_type=jnp.float32)
         + bn2_ref[...])
    a_ref[...] = (jnp.dot(u, wi_ref[...], preferred_element_type=jnp.float32)
                  + bm1_ref[...])
    b2_ref[...] = jnp.dot(u, wj_ref[...], preferred_element_type=jnp.float32)


def _post_node_stage(pp, cntp, wm2, bm2, g, b, wn1, bn1, wn2, bn2, wi, wj,
                     bm1, n, block):
    full = lambda s: pl.BlockSpec(s, lambda i: (0,) * len(s))
    return pl.pallas_call(
        _post_node_body,
        grid=(n // block,),
        in_specs=[
            pl.BlockSpec((2, block, H), lambda i: (0, i, 0)),
            pl.BlockSpec((2, block, VL), lambda i: (0, i, 0)),
            full((H, H)), full((1, H)), full((1, H)), full((1, H)),
            full((D, 2 * D)), full((1, 2 * D)), full((2 * D, D)),
            full((1, D)), full((D, H)), full((D, H)), full((1, H)),
        ],
        out_specs=[pl.BlockSpec((block, H), lambda i: (i, 0)),
                   pl.BlockSpec((block, H), lambda i: (i, 0))],
        out_shape=[jax.ShapeDtypeStruct((n, H), jnp.float32),
                   jax.ShapeDtypeStruct((n, H), jnp.float32)],
    )(pp, cntp, wm2, bm2.reshape(1, -1), g.reshape(1, -1), b.reshape(1, -1),
      wn1, bn1.reshape(1, -1), wn2, bn2.reshape(1, -1), wi, wj,
      bm1.reshape(1, -1))


def _sc_geometry(n, e):
    mesh = plsc.VectorSubcoreMesh(core_axis_name="c", subcore_axis_name="s")
    nc, ns = mesh.num_cores, mesh.num_subcores
    epw = e // (nc * ns)
    k = 80
    rows_pt = ((-(-n // ns)) + k - 1) // k * k
    n_pad = rows_pt * ns
    return mesh, nc, ns, epw, k, epw // k, rows_pt, n_pad, rows_pt // k


def _make_sc_edge_pass(n, e):
    mesh, nc, ns, _, _, _, _, _, _ = _sc_geometry(n, e)
    nw = nc * ns
    ke = 32
    total_ch = e // ke
    max_t = -(-total_ch // nw)
    nquads = (max_t + 4) // 4
    rows_pt = ((-(-n // ns)) + ke - 1) // ke * ke
    n_pad = rows_pt * ns
    nzc = rows_pt // ke

    out_type = [jax.ShapeDtypeStruct((nc, n_pad, H), jnp.float32)]
    scratch = [
        pltpu.VMEM_SHARED((n_pad, H), jnp.float32),
        pltpu.VMEM((4, ke), jnp.int32),
        pltpu.VMEM((4, ke), jnp.int32),
        pltpu.VMEM((ke, H), jnp.float32),
        pltpu.VMEM((ke, H), jnp.float32),
        pltpu.VMEM((ke, H), jnp.float32),
        pltpu.VMEM((ke, H), jnp.float32),
        pltpu.VMEM((ke // 2, H), jnp.int32),
        pltpu.VMEM((ke // 2, H), jnp.int32),
        pltpu.VMEM((ke, H), jnp.float32),
        pltpu.VMEM((ke, H), jnp.float32),
        pltpu.SemaphoreType.DMA,
        pltpu.SemaphoreType.DMA,
        pltpu.SemaphoreType.DMA,
        pltpu.SemaphoreType.DMA,
        pltpu.SemaphoreType.DMA,
        pltpu.SemaphoreType.DMA,
    ]

    def body(a_hbm, b_hbm, c_hbm, dst_hbm, src_hbm, p_hbm,
             s_sh, dir_, sir_, ab0, ab1, bb0, bb1, cb0, cb1, mb0, mb1,
             sem_i0, sem_i1, sem_g0, sem_g1, sem_s0, sem_s1):
        ab = (ab0, ab1)
        bb = (bb0, bb1)
        cb = (cb0, cb1)
        mb = (mb0, mb1)
        sem_i = (sem_i0, sem_i1)
        sem_g = (sem_g0, sem_g1)
        sem_s = (sem_s0, sem_s1)
        c_id = lax.axis_index("c")
        s_id = lax.axis_index("s")
        wid = c_id * ns + s_id
        row0 = s_id * rows_pt
        nt = lax.div(total_ch - wid + (nw - 1), nw)

        zeros16 = jnp.zeros((VL,), jnp.float32)

        def zfill(i, _):
            for j in range(H // VL):
                mb0[i, pl.ds(j * VL, VL)] = zeros16
            return 0
        lax.fori_loop(0, ke, zfill, 0)
        for j in range(nzc):
            pltpu.sync_copy(mb0, s_sh.at[pl.ds(row0 + j * ke, ke)])
        plsc.subcore_barrier()

        def eoff_of(t):
            return pl.multiple_of((wid + nw * t) * ke, 8)

        def issue_idx(t, q, sync=False):
            eoff = eoff_of(t)
            if sync:
                pltpu.sync_copy(dst_hbm.at[pl.ds(eoff, ke)], dir_.at[q])
                pltpu.sync_copy(src_hbm.at[pl.ds(eoff, ke)], sir_.at[q])
            else:
                pltpu.async_copy(dst_hbm.at[pl.ds(eoff, ke)], dir_.at[q],
                                 sem_i[q % 2])
                pltpu.async_copy(src_hbm.at[pl.ds(eoff, ke)], sir_.at[q],
                                 sem_i[q % 2])

        def wait_idx(q):
            pltpu.make_async_copy(dst_hbm.at[pl.ds(0, ke)], dir_.at[q],
                                  sem_i[q % 2]).wait()
            pltpu.make_async_copy(src_hbm.at[pl.ds(0, ke)], sir_.at[q],
                                  sem_i[q % 2]).wait()

        def issue_gathers(t, b, q):
            eoff2 = pl.multiple_of((wid + nw * t) * (ke // 2), 8)
            pltpu.async_copy(a_hbm.at[dir_.at[q]], ab[b], sem_g[b])
            pltpu.async_copy(b_hbm.at[sir_.at[q]], bb[b], sem_g[b])
            pltpu.async_copy(c_hbm.at[pl.ds(eoff2, ke // 2)], cb[b],
                             sem_g[b])

        def wait_gathers(b):
            pltpu.make_async_copy(a_hbm.at[dir_.at[0]], ab[b],
                                  sem_g[b]).wait()
            pltpu.make_async_copy(b_hbm.at[sir_.at[0]], bb[b],
                                  sem_g[b]).wait()
            pltpu.make_async_copy(c_hbm.at[pl.ds(0, ke // 2)], cb[b],
                                  sem_g[b]).wait()

        def wait_scatter(b):
            pltpu.make_async_copy(mb[b], s_sh.at[dir_.at[0]],
                                  sem_s[b]).wait()

        issue_idx(0, 0, sync=True)
        issue_idx(1, 1)
        issue_gathers(0, 0, 0)

        himask = jnp.full((VL,), -65536, jnp.int32)

        def quad(i4, _):
            for qq in range(4):
                t = 4 * i4 + qq
                b = qq % 2

                @pl.when(t < nt)
                def _():
                    @pl.when(t + 1 < nt)
                    def _():
                        wait_idx((qq + 1) % 4)
                        issue_gathers(t + 1, 1 - b, (qq + 1) % 4)
                    wait_gathers(b)

                    @pl.when(t >= 2)
                    def _():
                        wait_scatter(b)

                    def edge2(r2, _):
                        for hh in range(2):
                            r = 2 * r2 + hh
                            for j in range(H // 2 // VL):
                                ci = cb[b][r2, pl.ds(hh * (H // 2) + j * VL,
                                                     VL)]
                                lo = pl.ds(j * VL, VL)
                                hi = pl.ds(H // 2 + j * VL, VL)
                                mb[b][r, lo] = jnp.maximum(
                                    ab[b][r, lo] + bb[b][r, lo]
                                    + plsc.bitcast(ci << 16, jnp.float32),
                                    zeros16)
                                mb[b][r, hi] = jnp.maximum(
                                    ab[b][r, hi] + bb[b][r, hi]
                                    + plsc.bitcast(ci & himask, jnp.float32),
                                    zeros16)
                        return 0
                    lax.fori_loop(0, ke // 2, edge2, 0)

                    pltpu.async_copy(mb[b], s_sh.at[dir_.at[qq]], sem_s[b],
                                     add=True)

                    @pl.when(t + 2 < nt)
                    def _():
                        issue_idx(t + 2, (qq + 2) % 4)
            return 0
        lax.fori_loop(0, nquads, quad, 0)

        wait_scatter(0)
        wait_scatter(1)
        plsc.subcore_barrier()

        for j in range(nzc):
            rows = pl.ds(row0 + j * ke, ke)
            pltpu.sync_copy(s_sh.at[rows], mb0)
            pltpu.sync_copy(mb0, p_hbm.at[c_id, rows])

    return pl.kernel(body, out_type=out_type, mesh=mesh,
                     scratch_types=scratch,
                     compiler_params=pltpu.CompilerParams(
                         needs_layout_passes=False))


def _make_sc_cnt_pass(n, e):
    mesh, nc, ns, epw, k, nch, rows_pt, n_pad, nzc = _sc_geometry(n, e)

    out_type = [jax.ShapeDtypeStruct((nc, n_pad, VL), jnp.float32)]
    scratch = [
        pltpu.VMEM_SHARED((n_pad, VL), jnp.float32),
        pltpu.VMEM((k,), jnp.int32),
        pltpu.VMEM((k,), jnp.int32),
        pltpu.VMEM((k, VL), jnp.float32),
        pltpu.SemaphoreType.DMA,
        pltpu.SemaphoreType.DMA,
    ]

    def body(dst_hbm, cnt_hbm, cnt_sh, di0, di1, ones, sem0, sem1):
        di = (di0, di1)
        sem = (sem0, sem1)
        c_id = lax.axis_index("c")
        s_id = lax.axis_index("s")
        wid = c_id * ns + s_id
        row0 = s_id * rows_pt

        def fill(val):
            def f(i, _):
                ones[i, pl.ds(0, VL)] = jnp.full((VL,), val, jnp.float32)
                return 0
            lax.fori_loop(0, k, f, 0)

        def issue_idx(t, b):
            eoff = pl.multiple_of(wid * epw + t * k, 8)
            pltpu.async_copy(dst_hbm.at[pl.ds(eoff, k)], di[b], sem[b])

        fill(0.0)
        for j in range(nzc):
            pltpu.sync_copy(ones, cnt_sh.at[pl.ds(row0 + j * k, k)])
        fill(1.0)
        plsc.subcore_barrier()

        issue_idx(0, 0)
        issue_idx(1, 1)

        def pair(i2, _):
            for b in (0, 1):
                t = 2 * i2 + b

                @pl.when(t < nch)
                def _():
                    pltpu.make_async_copy(dst_hbm.at[pl.ds(0, k)], di[b],
                                          sem[b]).wait()
                    pltpu.sync_copy(ones, cnt_sh.at[di[b]], add=True)

                    @pl.when(t + 2 < nch)
                    def _():
                        issue_idx(t + 2, b)
            return 0
        lax.fori_loop(0, (nch + 2) // 2, pair, 0)
        plsc.subcore_barrier()

        for j in range(nzc):
            rows = pl.ds(row0 + j * k, k)
            pltpu.sync_copy(cnt_sh.at[rows], ones)
            pltpu.sync_copy(ones, cnt_hbm.at[c_id, rows])

    return pl.kernel(body, out_type=out_type, mesh=mesh,
                     scratch_types=scratch)


def kernel(x, edge_index, edge_attr, Wn1, bn1, Wn2, bn2, We1, be1, We2, be2,
           Wm1, bm1, Wm2, bm2, g, b, Wf, bf):
    n = x.shape[0]
    e = edge_index.shape[1]
    num_layers = Wn1.shape[0]
    src = edge_index[0]
    dst = edge_index[1]

    sc_pass = _make_sc_edge_pass(n, e)
    (cntp,) = _make_sc_cnt_pass(n, e)(dst)

    ea2 = edge_attr.reshape(e // 2, 2 * ED)
    wi = [Wm1[l][:D] for l in range(num_layers)]
    wj = [Wm1[l][D:2 * D] for l in range(num_layers)]
    we = [Wm1[l][2 * D:] for l in range(num_layers)]
    c0 = _edge_stage(ea2, We1[0], be1[0], We2[0] @ we[0], be2[0] @ we[0],
                     block=4000)
    a, bmat = _node_stage(x, Wn1[0], bn1[0], Wn2[0], bn2[0], wi[0], wj[0],
                          bm1[0], block=2000)
    (p,) = sc_pass(a, bmat, c0, dst, src)
    c1 = _edge_stage(ea2, We1[1], be1[1], We2[1] @ we[1], be2[1] @ we[1],
                     block=4000)
    a, bmat = _post_node_stage(p, cntp, Wm2[0], bm2[0], g[0], b[0],
                               Wn1[1], bn1[1], Wn2[1], bn2[1], wi[1], wj[1],
                               bm1[1], n=n, block=2000)
    (p,) = sc_pass(a, bmat, c1, dst, src)
    return _post_stage(p, cntp, Wm2[1], bm2[1], g[1], b[1], Wf, bf,
                       n=n, block=2000, final=True)
